# baseline (speedup 1.0000x reference)
import functools

import jax
import jax.numpy as jnp
from jax import lax
from jax.experimental import pallas as pl
from jax.experimental.pallas import tpu as pltpu
from jax.experimental.pallas import tpu_sc as plsc

BATCH = 16384
FIELDS = 26
EMBED_DIM = 32
ROW_PAD = 33
VOCAB = 1000000
WEIGHT = 0.5
LANES = 16
NUM_CORES = 2
NUM_SUBCORES = 16
NW = NUM_CORES * NUM_SUBCORES
ROWS_PER_W = BATCH // NW
CHUNK_ROWS = 64
NCHUNKS = ROWS_PER_W // CHUNK_ROWS
IDX_PER_CHUNK = CHUNK_ROWS * FIELDS
IDX_TILE = 128
NIDX_TILES = IDX_PER_CHUNK // IDX_TILE

TBLK = 512
R_FULL = (VOCAB // TBLK) * TBLK
NBLK = R_FULL // TBLK
TAIL = VOCAB - R_FULL
NPAIR = (-(-NBLK // NW) + 1) // 2


def _vt_body(vt_hbm, vtail_hbm, out_hbm,
             vin0, vin1, stage0, stage1, tailv, tstage,
             isem0, isem1, osem0, osem1):
    wid = lax.axis_index("s") * NUM_CORES + lax.axis_index("c")
    iota = lax.iota(jnp.int32, LANES)
    vins = (vin0, vin1)
    stages = (stage0, stage1)
    isems = (isem0, isem1)
    osems = (osem0, osem1)

    @pl.when(wid == 0)
    def _():
        pltpu.sync_copy(vtail_hbm, tailv)
        for r in range(TAIL):
            for h in range(2):
                v = tailv[pl.ds(r * EMBED_DIM + h * LANES, LANES)]
                plsc.store_scatter(
                    tstage, [jnp.full((LANES,), r * ROW_PAD + h * LANES,
                                      jnp.int32) + iota], v)
        pltpu.sync_copy(tstage, out_hbm.at[pl.ds(R_FULL * ROW_PAD,
                                                 TAIL * ROW_PAD)])

    nper = -(-NBLK // NW)

    def do_blk(i, carry):
        blk = wid + i * NW

        @pl.when(blk < NBLK)
        def _():
            pltpu.sync_copy(vt_hbm.at[:, pl.ds(blk * TBLK, TBLK)], vin0)

            def do_grp(g, carry2):
                rloc33 = (g * LANES + iota) * ROW_PAD
                for d in range(EMBED_DIM):
                    v = vin0[d, pl.ds(g * LANES, LANES)]
                    plsc.store_scatter(stage0, [rloc33 + d], v)
                return carry2

            lax.fori_loop(0, TBLK // LANES, do_grp, 0)
            pltpu.sync_copy(
                stage0,
                out_hbm.at[pl.ds(blk * TBLK * ROW_PAD, TBLK * ROW_PAD)])

        return carry

    lax.fori_loop(0, nper, do_blk, 0)


@jax.jit
def _vt_transpose(vt, vtail):
    mesh = plsc.VectorSubcoreMesh(core_axis_name="c", subcore_axis_name="s")
    f = functools.partial(
        pl.kernel,
        out_type=jax.ShapeDtypeStruct((VOCAB * ROW_PAD,), jnp.float32),
        mesh=mesh,
        compiler_params=pltpu.CompilerParams(
            use_tc_tiling_on_sc=True, needs_layout_passes=False),
        scratch_types=[
            pltpu.VMEM((EMBED_DIM, TBLK), jnp.float32),
            pltpu.VMEM((EMBED_DIM, TBLK), jnp.float32),
            pltpu.VMEM((TBLK * ROW_PAD,), jnp.float32),
            pltpu.VMEM((TBLK * ROW_PAD,), jnp.float32),
            pltpu.VMEM((TAIL * EMBED_DIM,), jnp.float32),
            pltpu.VMEM((TAIL * ROW_PAD,), jnp.float32),
            pltpu.SemaphoreType.DMA,
            pltpu.SemaphoreType.DMA,
            pltpu.SemaphoreType.DMA,
            pltpu.SemaphoreType.DMA,
        ],
    )(_vt_body)
    return f(vt, vtail)


def _fm_body(x_hbm, w0_hbm, w1_hbm, v_hbm, out_hbm,
             idx_v, rows_v, w1_v, out_v, w0_v, gsem, wsem):
    wid = lax.axis_index("s") * NUM_CORES + lax.axis_index("c")
    pltpu.sync_copy(w0_hbm, w0_v)
    w0vec = w0_v[...]
    w1_v[pl.ds(IDX_PER_CHUNK, LANES)] = jnp.zeros((LANES,), jnp.float32)
    iota = lax.iota(jnp.int32, LANES)
    mask_tail = (iota < (FIELDS - LANES)).astype(jnp.float32)

    def do_chunk(c, carry):
        row_base = wid * ROWS_PER_W + c * CHUNK_ROWS
        xoff = row_base * FIELDS
        pltpu.sync_copy(x_hbm.at[pl.ds(xoff, IDX_PER_CHUNK)], idx_v)
        copies = []
        for j in range(NIDX_TILES):
            copies.append(pltpu.async_copy(
                v_hbm.at[idx_v.at[pl.ds(j * IDX_TILE, IDX_TILE)]],
                rows_v.at[pl.ds(j * IDX_TILE, IDX_TILE)], gsem))
            copies.append(pltpu.async_copy(
                w1_hbm.at[idx_v.at[pl.ds(j * IDX_TILE, IDX_TILE)]],
                w1_v.at[pl.ds(j * IDX_TILE, IDX_TILE)], wsem))
        for cp in copies:
            cp.wait()

        def row_body(b, carry2):
            rbase = b * FIELDS
            acc0 = jnp.zeros((LANES,), jnp.float32)
            acc1 = jnp.zeros((LANES,), jnp.float32)
            sq0 = jnp.zeros((LANES,), jnp.float32)
            sq1 = jnp.zeros((LANES,), jnp.float32)
            for f in range(FIELDS):
                v0 = rows_v[rbase + f, pl.ds(0, LANES)]
                v1 = rows_v[rbase + f, pl.ds(LANES, LANES)]
                acc0 = acc0 + v0
                acc1 = acc1 + v1
                sq0 = sq0 + v0 * v0
                sq1 = sq1 + v1 * v1
            l0 = plsc.load_gather(w1_v, [rbase + iota])
            l1 = plsc.load_gather(w1_v, [rbase + LANES + iota]) * mask_tail
            lin = jnp.sum(l0 + l1)
            linv = jnp.full((LANES,), lin, jnp.float32) + w0vec
            out_v[b, pl.ds(0, LANES)] = linv + WEIGHT * (acc0 * acc0 + sq0)
            out_v[b, pl.ds(LANES, LANES)] = linv + WEIGHT * (acc1 * acc1 + sq1)
            return carry2

        lax.fori_loop(0, CHUNK_ROWS, row_body, 0)
        pltpu.sync_copy(out_v, out_hbm.at[pl.ds(row_base, CHUNK_ROWS)])
        return carry

    lax.fori_loop(0, NCHUNKS, do_chunk, 0)


@jax.jit
def _fm(x2, w0b, w1f, v2):
    mesh = plsc.VectorSubcoreMesh(core_axis_name="c", subcore_axis_name="s")
    f = functools.partial(
        pl.kernel,
        out_type=jax.ShapeDtypeStruct((BATCH, EMBED_DIM), jnp.float32),
        mesh=mesh,
        compiler_params=pltpu.CompilerParams(
            use_tc_tiling_on_sc=False, needs_layout_passes=False),
        scratch_types=[
            pltpu.VMEM((IDX_PER_CHUNK,), jnp.int32),
            pltpu.VMEM((IDX_PER_CHUNK, ROW_PAD), jnp.float32),
            pltpu.VMEM((IDX_PER_CHUNK + LANES,), jnp.float32),
            pltpu.VMEM((CHUNK_ROWS, EMBED_DIM), jnp.float32),
            pltpu.VMEM((LANES,), jnp.float32),
            pltpu.SemaphoreType.DMA,
            pltpu.SemaphoreType.DMA,
        ],
    )(_fm_body)
    return f(x2, w0b, w1f, v2)


def kernel(x, W0, W1, V):
    x2 = x.reshape(BATCH * FIELDS).astype(jnp.int32)
    w0b = jnp.broadcast_to(W0.astype(jnp.float32), (LANES,))
    w1f = W1.reshape(-1)
    vt = V.T
    vtail = lax.slice(V, (R_FULL, 0), (VOCAB, EMBED_DIM)).reshape(-1)
    vlin = _vt_transpose(vt, vtail)
    v2 = vlin.reshape(VOCAB, ROW_PAD)
    return _fm(x2, w0b, w1f, v2)


# speedup vs baseline: 2.1112x; 2.1112x over previous
import functools

import jax
import jax.numpy as jnp
from jax import lax
from jax.experimental import pallas as pl
from jax.experimental.pallas import tpu as pltpu
from jax.experimental.pallas import tpu_sc as plsc

BATCH = 16384
FIELDS = 26
EMBED_DIM = 32
WEIGHT = 0.5
LANES = 16
NUM_CORES = 2
NUM_SUBCORES = 16
NW = NUM_CORES * NUM_SUBCORES
ROWS_PER_W = BATCH // NW
CHUNK_ROWS = 64
NCHUNKS = ROWS_PER_W // CHUNK_ROWS
IDX_PER_CHUNK = CHUNK_ROWS * FIELDS
IDX_TILE = 128
NIDX_TILES = IDX_PER_CHUNK // IDX_TILE


DT_ROWS = 320
DT_NBLK = 1000000 // DT_ROWS
VOCAB = 1000000


def _dt_body(v_hbm, out_hbm, vin, stage):
    wid = lax.axis_index("s") * NUM_CORES + lax.axis_index("c")
    nper = -(-DT_NBLK // NW)

    def do_blk(i, carry):
        blk = wid + i * NW

        @pl.when(blk < DT_NBLK)
        def _():
            r0 = blk * DT_ROWS
            pltpu.sync_copy(v_hbm.at[pl.ds(r0, DT_ROWS), :], vin)

            def do_row(r, carry2):
                stage[pl.ds(r * EMBED_DIM, LANES)] = vin[r, pl.ds(0, LANES)]
                stage[pl.ds(r * EMBED_DIM + LANES, LANES)] = (
                    vin[r, pl.ds(LANES, LANES)])
                return carry2

            lax.fori_loop(0, DT_ROWS, do_row, 0)
            pltpu.sync_copy(stage, out_hbm.at[pl.ds(r0 * EMBED_DIM,
                                                    DT_ROWS * EMBED_DIM)])

        return carry

    lax.fori_loop(0, nper, do_blk, 0)


@jax.jit
def _detile(V):
    mesh = plsc.VectorSubcoreMesh(core_axis_name="c", subcore_axis_name="s")
    f = functools.partial(
        pl.kernel,
        out_type=jax.ShapeDtypeStruct((VOCAB * EMBED_DIM,), jnp.float32),
        mesh=mesh,
        compiler_params=pltpu.CompilerParams(
            use_tc_tiling_on_sc=True, needs_layout_passes=False),
        scratch_types=[
            pltpu.VMEM((DT_ROWS, EMBED_DIM), jnp.float32),
            pltpu.VMEM((DT_ROWS * EMBED_DIM,), jnp.float32),
        ],
    )(_dt_body)
    return f(V)


def _fm_body(x_hbm, w0_hbm, w1_hbm, v_hbm, out_hbm,
             idx_v, rows_v, w1_v, out_v, w0_v, gsem, wsem):
    wid = lax.axis_index("s") * NUM_CORES + lax.axis_index("c")
    pltpu.sync_copy(w0_hbm, w0_v)
    w0vec = w0_v[...]
    w1_v[pl.ds(IDX_PER_CHUNK, LANES)] = jnp.zeros((LANES,), jnp.float32)
    iota = lax.iota(jnp.int32, LANES)
    mask_tail = (iota < (FIELDS - LANES)).astype(jnp.float32)

    def do_chunk(c, carry):
        row_base = wid * ROWS_PER_W + c * CHUNK_ROWS
        xoff = (wid * ROWS_PER_W + c * CHUNK_ROWS) * FIELDS
        pltpu.sync_copy(x_hbm.at[pl.ds(xoff, IDX_PER_CHUNK)], idx_v)
        copies = []
        for j in range(NIDX_TILES):
            copies.append(pltpu.async_copy(
                v_hbm.at[idx_v.at[pl.ds(j * IDX_TILE, IDX_TILE)]],
                rows_v.at[pl.ds(j * IDX_TILE, IDX_TILE)], gsem))
            copies.append(pltpu.async_copy(
                w1_hbm.at[idx_v.at[pl.ds(j * IDX_TILE, IDX_TILE)]],
                w1_v.at[pl.ds(j * IDX_TILE, IDX_TILE)], wsem))
        for cp in copies:
            cp.wait()

        def row_body(b, carry2):
            rbase = b * FIELDS
            acc0 = jnp.zeros((LANES,), jnp.float32)
            acc1 = jnp.zeros((LANES,), jnp.float32)
            sq0 = jnp.zeros((LANES,), jnp.float32)
            sq1 = jnp.zeros((LANES,), jnp.float32)
            for f in range(FIELDS):
                v0 = rows_v[rbase + f, pl.ds(0, LANES)]
                v1 = rows_v[rbase + f, pl.ds(LANES, LANES)]
                acc0 = acc0 + v0
                acc1 = acc1 + v1
                sq0 = sq0 + v0 * v0
                sq1 = sq1 + v1 * v1
            l0 = plsc.load_gather(w1_v, [rbase + iota])
            l1 = plsc.load_gather(w1_v, [rbase + LANES + iota]) * mask_tail
            lin = jnp.sum(l0 + l1)
            linv = jnp.full((LANES,), lin, jnp.float32) + w0vec
            out_v[b, pl.ds(0, LANES)] = linv + WEIGHT * (acc0 * acc0 + sq0)
            out_v[b, pl.ds(LANES, LANES)] = linv + WEIGHT * (acc1 * acc1 + sq1)
            return carry2

        lax.fori_loop(0, CHUNK_ROWS, row_body, 0)
        pltpu.sync_copy(out_v, out_hbm.at[pl.ds(row_base, CHUNK_ROWS)])
        return carry

    lax.fori_loop(0, NCHUNKS, do_chunk, 0)


@jax.jit
def _fm(x2, w0b, w1f, V):
    mesh = plsc.VectorSubcoreMesh(core_axis_name="c", subcore_axis_name="s")
    f = functools.partial(
        pl.kernel,
        out_type=jax.ShapeDtypeStruct((BATCH, EMBED_DIM), jnp.float32),
        mesh=mesh,
        compiler_params=pltpu.CompilerParams(
            use_tc_tiling_on_sc=False, needs_layout_passes=False),
        scratch_types=[
            pltpu.VMEM((IDX_PER_CHUNK,), jnp.int32),
            pltpu.VMEM((IDX_PER_CHUNK, EMBED_DIM), jnp.float32),
            pltpu.VMEM((IDX_PER_CHUNK + LANES,), jnp.float32),
            pltpu.VMEM((CHUNK_ROWS, EMBED_DIM), jnp.float32),
            pltpu.VMEM((LANES,), jnp.float32),
            pltpu.SemaphoreType.DMA,
            pltpu.SemaphoreType.DMA,
        ],
    )(_fm_body)
    return f(x2, w0b, w1f, V)


def kernel(x, W0, W1, V):
    x2 = x.reshape(BATCH * FIELDS).astype(jnp.int32)
    w0b = jnp.broadcast_to(W0.astype(jnp.float32), (LANES,))
    w1f = W1.reshape(-1)
    v2 = _detile(V).reshape(VOCAB, EMBED_DIM)
    return _fm(x2, w0b, w1f, v2)


# speedup vs baseline: 3.2039x; 1.5175x over previous
import functools

import jax
import jax.numpy as jnp
from jax import lax
from jax.experimental import pallas as pl
from jax.experimental.pallas import tpu as pltpu
from jax.experimental.pallas import tpu_sc as plsc

BATCH = 16384
FIELDS = 26
EMBED_DIM = 32
WEIGHT = 0.5
LANES = 16
NUM_CORES = 2
NUM_SUBCORES = 16
NW = NUM_CORES * NUM_SUBCORES
ROWS_PER_W = BATCH // NW
CHUNK_ROWS = 64
NCHUNKS = ROWS_PER_W // CHUNK_ROWS
IDX_PER_CHUNK = CHUNK_ROWS * FIELDS
IDX_TILE = 128
NIDX_TILES = IDX_PER_CHUNK // IDX_TILE


def _fm_body(x_hbm, w0_hbm, w1_hbm, v_hbm, out_hbm,
             idx_v, rows_v, w1_v, out_v, w0_v, gsem, wsem):
    wid = lax.axis_index("s") * NUM_CORES + lax.axis_index("c")
    pltpu.sync_copy(w0_hbm, w0_v)
    w0vec = w0_v[...]
    w1_v[pl.ds(IDX_PER_CHUNK, LANES)] = jnp.zeros((LANES,), jnp.float32)
    iota = lax.iota(jnp.int32, LANES)
    mask_tail = (iota < (FIELDS - LANES)).astype(jnp.float32)

    def do_chunk(c, carry):
        row_base = wid * ROWS_PER_W + c * CHUNK_ROWS
        xoff = (wid * ROWS_PER_W + c * CHUNK_ROWS) * FIELDS
        pltpu.sync_copy(x_hbm.at[pl.ds(xoff, IDX_PER_CHUNK)], idx_v)
        copies = []
        for j in range(NIDX_TILES):
            copies.append(pltpu.async_copy(
                v_hbm.at[idx_v.at[pl.ds(j * IDX_TILE, IDX_TILE)]],
                rows_v.at[pl.ds(j * IDX_TILE, IDX_TILE)], gsem))
            copies.append(pltpu.async_copy(
                w1_hbm.at[idx_v.at[pl.ds(j * IDX_TILE, IDX_TILE)]],
                w1_v.at[pl.ds(j * IDX_TILE, IDX_TILE)], wsem))
        for cp in copies:
            cp.wait()

        def row_body(b, carry2):
            rbase = b * FIELDS
            acc0 = jnp.zeros((LANES,), jnp.float32)
            acc1 = jnp.zeros((LANES,), jnp.float32)
            sq0 = jnp.zeros((LANES,), jnp.float32)
            sq1 = jnp.zeros((LANES,), jnp.float32)
            for f in range(FIELDS):
                v0 = rows_v[rbase + f, pl.ds(0, LANES)]
                v1 = rows_v[rbase + f, pl.ds(LANES, LANES)]
                acc0 = acc0 + v0
                acc1 = acc1 + v1
                sq0 = sq0 + v0 * v0
                sq1 = sq1 + v1 * v1
            l0 = plsc.load_gather(w1_v, [rbase + iota])
            l1 = plsc.load_gather(w1_v, [rbase + LANES + iota]) * mask_tail
            lin = jnp.sum(l0 + l1)
            linv = jnp.full((LANES,), lin, jnp.float32) + w0vec
            out_v[b, pl.ds(0, LANES)] = linv + WEIGHT * (acc0 * acc0 + sq0)
            out_v[b, pl.ds(LANES, LANES)] = linv + WEIGHT * (acc1 * acc1 + sq1)
            return carry2

        lax.fori_loop(0, CHUNK_ROWS, row_body, 0)
        pltpu.sync_copy(out_v, out_hbm.at[pl.ds(row_base, CHUNK_ROWS)])
        return carry

    lax.fori_loop(0, NCHUNKS, do_chunk, 0)


@jax.jit
def _fm(x2, w0b, w1f, V):
    mesh = plsc.VectorSubcoreMesh(core_axis_name="c", subcore_axis_name="s")
    f = functools.partial(
        pl.kernel,
        out_type=jax.ShapeDtypeStruct((BATCH, EMBED_DIM), jnp.float32),
        mesh=mesh,
        compiler_params=pltpu.CompilerParams(
            use_tc_tiling_on_sc=False, needs_layout_passes=False),
        scratch_types=[
            pltpu.VMEM((IDX_PER_CHUNK,), jnp.int32),
            pltpu.VMEM((IDX_PER_CHUNK, EMBED_DIM), jnp.float32),
            pltpu.VMEM((IDX_PER_CHUNK + LANES,), jnp.float32),
            pltpu.VMEM((CHUNK_ROWS, EMBED_DIM), jnp.float32),
            pltpu.VMEM((LANES,), jnp.float32),
            pltpu.SemaphoreType.DMA,
            pltpu.SemaphoreType.DMA,
        ],
    )(_fm_body)
    return f(x2, w0b, w1f, V)


def kernel(x, W0, W1, V):
    x2 = x.reshape(BATCH * FIELDS).astype(jnp.int32)
    w0b = jnp.broadcast_to(W0.astype(jnp.float32), (LANES,))
    w1f = W1.reshape(-1)
    return _fm(x2, w0b, w1f, V)
